# Initial kernel scaffold; baseline (speedup 1.0000x reference)
import jax
import jax.numpy as jnp
from jax.experimental import pallas as pl


def kernel(logits, gamma, beta):
    raise NotImplementedError("write your pallas kernel here")


# speedup vs baseline: 2.9342x; 2.9342x over previous
import jax
import jax.numpy as jnp
from jax import lax
from jax.experimental import pallas as pl

SEQ = 1000000
C = 4
LANES = 128
BL = 131072
K = BL // LANES
GRID = (SEQ + BL - 1) // BL

_TINY = 1.1754943508222875e-38
_ROTS = ((13, 15, 26, 6), (17, 29, 16, 24))
_KS0 = 0
_KS1 = 42
_KS2 = (0x1BD11BDA ^ _KS0 ^ _KS1) & 0xFFFFFFFF


def _stats_kernel(x_ref, sum_ref, sq_ref):
    i = pl.program_id(0)

    @pl.when(i == 0)
    def _():
        sum_ref[...] = jnp.zeros_like(sum_ref)
        sq_ref[...] = jnp.zeros_like(sq_ref)

    x = x_ref[...].reshape(C, K, LANES)

    @pl.when(i < GRID - 1)
    def _():
        sum_ref[0:C, :] += jnp.sum(x, axis=1)
        sq_ref[0:C, :] += jnp.sum(x * x, axis=1)

    @pl.when(i == GRID - 1)
    def _():
        k = lax.broadcasted_iota(jnp.int32, (C, K, LANES), 1)
        l = lax.broadcasted_iota(jnp.int32, (C, K, LANES), 2)
        valid = (i * BL + k * LANES + l) < SEQ
        xv = jnp.where(valid, x, 0.0)
        sum_ref[0:C, :] += jnp.sum(xv, axis=1)
        sq_ref[0:C, :] += jnp.sum(xv * xv, axis=1)


def _threefry_bits(flat_u32):
    ks = (jnp.uint32(_KS0), jnp.uint32(_KS1), jnp.uint32(_KS2))
    x0 = jnp.zeros_like(flat_u32) + ks[0]
    x1 = flat_u32 + ks[1]

    def rotl(v, d):
        return (v << jnp.uint32(d)) | (v >> jnp.uint32(32 - d))

    for i in range(5):
        for r in _ROTS[i % 2]:
            x0 = x0 + x1
            x1 = rotl(x1, r)
            x1 = x1 ^ x0
        x0 = x0 + ks[(i + 1) % 3]
        x1 = x1 + ks[(i + 2) % 3] + jnp.uint32(i + 1)
    return x0 ^ x1


def _main_kernel(x_ref, aux_ref, oh_ref, sc_ref):
    i = pl.program_id(0)
    x = x_ref[...].reshape(C, K, LANES)
    aux = aux_ref[...]
    scale = aux[0:C, :].reshape(C, 1, LANES)
    shift = aux[C:2 * C, :].reshape(C, 1, LANES)
    scaled = x * scale + shift
    sc_ref[...] = scaled.reshape(C, BL)

    e = jnp.exp(scaled)
    se = (e[0] + e[1]) + (e[2] + e[3])
    probs = e / se[None, :, :]

    c = lax.broadcasted_iota(jnp.int32, (C, K, LANES), 0)
    k = lax.broadcasted_iota(jnp.int32, (C, K, LANES), 1)
    l = lax.broadcasted_iota(jnp.int32, (C, K, LANES), 2)
    flat = (i * BL + k * LANES + l) * C + c
    bits = _threefry_bits(flat.astype(jnp.uint32))
    fb = (bits >> jnp.uint32(9)) | jnp.uint32(0x3F800000)
    u01 = lax.bitcast_convert_type(fb, jnp.float32) - jnp.float32(1.0)
    tiny = jnp.float32(_TINY)
    u = jnp.maximum(tiny, u01 + tiny)
    g = -jnp.log(-jnp.log(u))

    y = g + scaled
    gm = jnp.maximum(jnp.maximum(y[0], y[1]), jnp.maximum(y[2], y[3]))
    cand = jnp.where(y == gm[None, :, :], c, 8)
    kmin = jnp.minimum(jnp.minimum(cand[0], cand[1]),
                       jnp.minimum(cand[2], cand[3]))
    hard = (c == kmin[None, :, :]).astype(jnp.float32)

    oh_ref[...] = ((hard - probs) + probs).reshape(C, BL)


def kernel(logits, gamma, beta):
    xt = logits.T

    sums, sqs = pl.pallas_call(
        _stats_kernel,
        grid=(GRID,),
        in_specs=[pl.BlockSpec((C, BL), lambda i: (0, i))],
        out_specs=[
            pl.BlockSpec((8, LANES), lambda i: (0, 0)),
            pl.BlockSpec((8, LANES), lambda i: (0, 0)),
        ],
        out_shape=[
            jax.ShapeDtypeStruct((8, LANES), jnp.float32),
            jax.ShapeDtypeStruct((8, LANES), jnp.float32),
        ],
    )(xt)

    s = sums[0:C, :].sum(axis=1)
    q = sqs[0:C, :].sum(axis=1)
    n = jnp.float32(SEQ)
    mu = s / n
    var = q / n - mu * mu
    inv = lax.rsqrt(var + jnp.float32(1e-5))
    scale = gamma * inv
    shift = beta - mu * scale
    aux = jnp.concatenate([
        jnp.broadcast_to(scale[:, None], (C, LANES)),
        jnp.broadcast_to(shift[:, None], (C, LANES)),
    ], axis=0)

    oh, sc = pl.pallas_call(
        _main_kernel,
        grid=(GRID,),
        in_specs=[
            pl.BlockSpec((C, BL), lambda i: (0, i)),
            pl.BlockSpec((2 * C, LANES), lambda i: (0, 0)),
        ],
        out_specs=[
            pl.BlockSpec((C, BL), lambda i: (0, i)),
            pl.BlockSpec((C, BL), lambda i: (0, i)),
        ],
        out_shape=[
            jax.ShapeDtypeStruct((C, SEQ), jnp.float32),
            jax.ShapeDtypeStruct((C, SEQ), jnp.float32),
        ],
    )(xt, aux)

    return oh.T, sc.T
